# baseline (speedup 1.0000x reference)
import functools

import jax
import jax.numpy as jnp
from jax import lax
from jax.experimental import pallas as pl
from jax.experimental.pallas import tpu as pltpu
from jax.experimental.pallas import tpu_sc as plsc

N = 10000
E = 320000
C = 128
KHOPS = 3
NC = 2
NS = 16
NW = NC * NS
CHUNK = 128
CW = 80
EP = NW * CW * CHUNK
NROWS = EP // CHUNK
NACC = 10112
RPS = NACC // NS
BLK = 1000
GRID = N // BLK

_mesh = plsc.VectorSubcoreMesh(core_axis_name="c", subcore_axis_name="s")


NRB = 2
NIB = 4
SPLIT = 1
SUB = CHUNK // SPLIT


@functools.partial(
    pl.kernel,
    out_type=jax.ShapeDtypeStruct((NC, NACC, C), jnp.float32),
    mesh=_mesh,
    scratch_types=(
        [pltpu.VMEM((CHUNK,), jnp.int32)] * NIB
        + [pltpu.VMEM((CHUNK,), jnp.int32)] * NIB
        + [pltpu.VMEM((CHUNK, C), jnp.float32)] * NRB
        + [pltpu.SemaphoreType.DMA] * NIB
        + [pltpu.SemaphoreType.DMA] * NIB
        + [pltpu.SemaphoreType.DMA] * (NRB * SPLIT)
        + [pltpu.SemaphoreType.DMA] * NRB
        + [pltpu.VMEM_SHARED((NACC, C), jnp.float32)]
    ),
)
def _sc_hop(g_hbm, src_hbm, dst_hbm, zeros_hbm, out_hbm, *bufs):
    srcs = bufs[0:NIB]
    dsts = bufs[NIB:2 * NIB]
    rows = bufs[2 * NIB:2 * NIB + NRB]
    isem = bufs[2 * NIB + NRB:3 * NIB + NRB]
    dsem = bufs[3 * NIB + NRB:4 * NIB + NRB]
    _g0 = 4 * NIB + NRB
    gsem = bufs[_g0:_g0 + NRB * SPLIT]
    ssem = bufs[_g0 + NRB * SPLIT:_g0 + NRB * SPLIT + NRB]
    acc_sh = bufs[_g0 + NRB * SPLIT + NRB]
    c = lax.axis_index("c")
    s = lax.axis_index("s")
    w = c * NS + s
    base = w * CW
    pltpu.sync_copy(zeros_hbm, acc_sh.at[pl.ds(s * RPS, RPS)])
    plsc.subcore_barrier()

    def load_idx(j, ib):
        pltpu.async_copy(src_hbm.at[base + j], srcs[ib], isem[ib])
        pltpu.async_copy(dst_hbm.at[base + j], dsts[ib], dsem[ib])

    def start_gather(j, ib, rb):
        pltpu.make_async_copy(src_hbm.at[base + j], srcs[ib], isem[ib]).wait()
        for k in range(SPLIT):
            pltpu.async_copy(
                g_hbm.at[srcs[ib].at[pl.ds(k * SUB, SUB)]],
                rows[rb].at[pl.ds(k * SUB, SUB)], gsem[rb * SPLIT + k])

    def wait_gather(rb):
        for k in range(SPLIT):
            pltpu.make_async_copy(
                g_hbm.at[srcs[0].at[pl.ds(0, SUB)]],
                rows[rb].at[pl.ds(k * SUB, SUB)], gsem[rb * SPLIT + k]).wait()

    load_idx(0, 0)
    load_idx(1, 1)
    start_gather(0, 0, 0)

    @pl.loop(0, CW // NIB)
    def _(gi):
        for b0 in range(NIB):
            rb = b0 % NRB
            j = gi * NIB + b0
            wait_gather(rb)
            pltpu.make_async_copy(dst_hbm.at[base + j], dsts[b0],
                                  dsem[b0]).wait()
            pltpu.async_copy(rows[rb], acc_sh.at[dsts[b0]], ssem[rb],
                             add=True)

            @pl.when(j >= 1)
            def _():
                pltpu.make_async_copy(rows[1 - rb], acc_sh.at[dsts[b0]],
                                      ssem[1 - rb]).wait()

            @pl.when(j + 1 < CW)
            def _():
                start_gather(j + 1, (b0 + 1) % NIB, 1 - rb)

            @pl.when(j + 2 < CW)
            def _():
                load_idx(j + 2, (b0 + 2) % NIB)

    pltpu.make_async_copy(rows[(CW - 1) % NRB], acc_sh.at[dsts[0]],
                          ssem[(CW - 1) % NRB]).wait()

    plsc.subcore_barrier()
    pltpu.sync_copy(acc_sh.at[pl.ds(s * RPS, RPS)],
                    out_hbm.at[c, pl.ds(s * RPS, RPS)])


def _dis_body(p0_ref, p1_ref, dis_ref):
    deg = (p0_ref[...] + p1_ref[...])[0, :, 0:1]
    dis = jnp.where(deg > 0, lax.rsqrt(jnp.maximum(deg, 1e-12)), 0.0)
    dis_ref[...] = jnp.broadcast_to(dis, (BLK, C))


def _tc_dis(p):
    return pl.pallas_call(
        _dis_body,
        grid=(GRID,),
        in_specs=[
            pl.BlockSpec((1, BLK, C), lambda i: (0, i, 0)),
            pl.BlockSpec((1, BLK, C), lambda i: (1, i, 0)),
        ],
        out_specs=pl.BlockSpec((BLK, C), lambda i: (i, 0)),
        out_shape=jax.ShapeDtypeStruct((N, C), jnp.float32),
    )(p, p)


def _mm_body(x_ref, w_ref, b_ref, dis_ref, h_ref, g_ref):
    h = lax.dot_general(x_ref[...], w_ref[...], (((1,), (0,)), ((), ())),
                        precision=lax.Precision.HIGHEST,
                        preferred_element_type=jnp.float32)
    h = h + b_ref[...]
    h_ref[...] = h
    g_ref[...] = h * dis_ref[...]


def _tc_matmul_scale(x, w, b, dis):
    return pl.pallas_call(
        _mm_body,
        grid=(GRID,),
        in_specs=[
            pl.BlockSpec((BLK, C), lambda i: (i, 0)),
            pl.BlockSpec((C, C), lambda i: (0, 0)),
            pl.BlockSpec((1, C), lambda i: (0, 0)),
            pl.BlockSpec((BLK, C), lambda i: (i, 0)),
        ],
        out_specs=[
            pl.BlockSpec((BLK, C), lambda i: (i, 0)),
            pl.BlockSpec((BLK, C), lambda i: (i, 0)),
        ],
        out_shape=[
            jax.ShapeDtypeStruct((N, C), jnp.float32),
            jax.ShapeDtypeStruct((N, C), jnp.float32),
        ],
    )(x, w, b, dis)


def _comb_body(p0_ref, p1_ref, dis_ref, h_ref, g_ref):
    dis = dis_ref[...]
    h = (p0_ref[...] + p1_ref[...])[0] * dis
    h_ref[...] = h
    g_ref[...] = h * dis


def _tc_combine(p, dis):
    return pl.pallas_call(
        _comb_body,
        grid=(GRID,),
        in_specs=[
            pl.BlockSpec((1, BLK, C), lambda i: (0, i, 0)),
            pl.BlockSpec((1, BLK, C), lambda i: (1, i, 0)),
            pl.BlockSpec((BLK, C), lambda i: (i, 0)),
        ],
        out_specs=[
            pl.BlockSpec((BLK, C), lambda i: (i, 0)),
            pl.BlockSpec((BLK, C), lambda i: (i, 0)),
        ],
        out_shape=[
            jax.ShapeDtypeStruct((N, C), jnp.float32),
            jax.ShapeDtypeStruct((N, C), jnp.float32),
        ],
    )(p, p, dis)


def _se_mix(h0, h1, h2, h3, se1_ref, se2_ref):
    hops = (h0, h1, h2, h3)
    z = [jnp.mean(h, axis=1, keepdims=True) for h in hops]
    u = [jnp.maximum(sum(z[i] * se1_ref[i, k] for i in range(4)), 0.0)
         for k in range(4)]
    svals = [jax.nn.sigmoid(sum(u[k] * se2_ref[k, j] for k in range(4)))
             for j in range(4)]
    return sum(svals[j] * hops[j] for j in range(4))


def _se0_body(h0_ref, h1_ref, h2_ref, p0_ref, p1_ref, dis_ref, se1_ref,
              se2_ref, g_ref, b_ref, out_ref):
    h3 = (p0_ref[...] + p1_ref[...])[0] * dis_ref[...]
    y = _se_mix(h0_ref[...], h1_ref[...], h2_ref[...], h3, se1_ref, se2_ref)
    mu = jnp.mean(y, axis=1, keepdims=True)
    var = jnp.mean(y * y, axis=1, keepdims=True) - mu * mu
    y = (y - mu) * lax.rsqrt(var + 1e-5) * g_ref[...] + b_ref[...]
    out_ref[...] = jnp.clip(y, 0.0, 6.0)


def _se1_body(h0_ref, h1_ref, h2_ref, p0_ref, p1_ref, dis_ref, se1_ref,
              se2_ref, out_ref):
    h3 = (p0_ref[...] + p1_ref[...])[0] * dis_ref[...]
    y = _se_mix(h0_ref[...], h1_ref[...], h2_ref[...], h3, se1_ref, se2_ref)
    m = jnp.max(y, axis=1, keepdims=True)
    ex = jnp.exp(y - m)
    out_ref[...] = (y - m) - jnp.log(jnp.sum(ex, axis=1, keepdims=True))


_ROWSPEC = pl.BlockSpec((BLK, C), lambda i: (i, 0))
_PSPEC0 = pl.BlockSpec((1, BLK, C), lambda i: (0, i, 0))
_PSPEC1 = pl.BlockSpec((1, BLK, C), lambda i: (1, i, 0))
_SMEMSPEC = pl.BlockSpec(memory_space=pltpu.SMEM)
_VECSPEC = pl.BlockSpec((1, C), lambda i: (0, 0))


def _tc_se0(h0, h1, h2, p, dis, se1, se2, ln_g, ln_b):
    return pl.pallas_call(
        _se0_body,
        grid=(GRID,),
        in_specs=[_ROWSPEC, _ROWSPEC, _ROWSPEC, _PSPEC0, _PSPEC1, _ROWSPEC,
                  _SMEMSPEC, _SMEMSPEC, _VECSPEC, _VECSPEC],
        out_specs=_ROWSPEC,
        out_shape=jax.ShapeDtypeStruct((N, C), jnp.float32),
    )(h0, h1, h2, p, p, dis, se1, se2, ln_g, ln_b)


def _tc_se1(h0, h1, h2, p, dis, se1, se2):
    return pl.pallas_call(
        _se1_body,
        grid=(GRID,),
        in_specs=[_ROWSPEC, _ROWSPEC, _ROWSPEC, _PSPEC0, _PSPEC1, _ROWSPEC,
                  _SMEMSPEC, _SMEMSPEC],
        out_specs=_ROWSPEC,
        out_shape=jax.ShapeDtypeStruct((N, C), jnp.float32),
    )(h0, h1, h2, p, p, dis, se1, se2)


def kernel(n_feat, edge_index, W0, b0, se1_0, se2_0, ln_g, ln_b, W1, b1,
           se1_1, se2_1):
    src = edge_index[0]
    dst = edge_index[1]
    pad = EP - E
    order = jnp.argsort(src)
    srcp = jnp.concatenate([src[order], jnp.zeros((pad,), jnp.int32)])
    dstp = jnp.concatenate([dst[order], jnp.full((pad,), N, jnp.int32)])
    srcR = srcp.reshape(NROWS, CHUNK)
    dstR = dstp.reshape(NROWS, CHUNK)
    zerosA = jnp.zeros((RPS, C), jnp.float32)
    onesN = jnp.ones((N, C), jnp.float32)
    b0r = b0.reshape(1, C)
    b1r = b1.reshape(1, C)
    ln_gr = ln_g.reshape(1, C)
    ln_br = ln_b.reshape(1, C)

    pdeg = _sc_hop(onesN, srcR, dstR, zerosA)
    dis = _tc_dis(pdeg)

    def layer(x, w, b):
        h0, g = _tc_matmul_scale(x, w, b, dis)
        p = _sc_hop(g, srcR, dstR, zerosA)
        h1, g = _tc_combine(p, dis)
        p = _sc_hop(g, srcR, dstR, zerosA)
        h2, g = _tc_combine(p, dis)
        p = _sc_hop(g, srcR, dstR, zerosA)
        return h0, h1, h2, p

    h0, h1, h2, p = layer(n_feat, W0, b0r)
    x1 = _tc_se0(h0, h1, h2, p, dis, se1_0, se2_0, ln_gr, ln_br)
    h0, h1, h2, p = layer(x1, W1, b1r)
    return _tc_se1(h0, h1, h2, p, dis, se1_1, se2_1)


# speedup vs baseline: 1.2988x; 1.2988x over previous
import functools

import jax
import jax.numpy as jnp
from jax import lax
from jax.experimental import pallas as pl
from jax.experimental.pallas import tpu as pltpu
from jax.experimental.pallas import tpu_sc as plsc

N = 10000
E = 320000
C = 128
KHOPS = 3
NC = 2
NS = 16
NW = NC * NS
CHUNK = 128
CW = 80
EP = NW * CW * CHUNK
NROWS = EP // CHUNK
NACC = 10112
RPS = NACC // NS
BLK = 1000
GRID = N // BLK

_mesh = plsc.VectorSubcoreMesh(core_axis_name="c", subcore_axis_name="s")


NRB = 3
CWU = CW - CW % NRB


@functools.partial(
    pl.kernel,
    out_type=jax.ShapeDtypeStruct((NC, NACC, C), jnp.float32),
    mesh=_mesh,
    scratch_types=(
        [pltpu.VMEM((CHUNK,), jnp.int32)] * NRB
        + [pltpu.VMEM((CHUNK,), jnp.int32)] * NRB
        + [pltpu.VMEM((CHUNK, C), jnp.float32)] * NRB
        + [pltpu.SemaphoreType.DMA] * NRB
        + [pltpu.SemaphoreType.DMA] * NRB
        + [pltpu.SemaphoreType.DMA] * NRB
        + [pltpu.SemaphoreType.DMA] * NRB
        + [pltpu.VMEM_SHARED((NACC, C), jnp.float32)]
    ),
)
def _sc_hop(g_hbm, src_hbm, dst_hbm, zeros_hbm, out_hbm, *bufs):
    srcs = bufs[0:NRB]
    dsts = bufs[NRB:2 * NRB]
    rows = bufs[2 * NRB:3 * NRB]
    isem = bufs[3 * NRB:4 * NRB]
    dsem = bufs[4 * NRB:5 * NRB]
    gsem = bufs[5 * NRB:6 * NRB]
    ssem = bufs[6 * NRB:7 * NRB]
    acc_sh = bufs[7 * NRB]
    c = lax.axis_index("c")
    s = lax.axis_index("s")
    w = c * NS + s
    base = w * CW
    pltpu.sync_copy(zeros_hbm, acc_sh.at[pl.ds(s * RPS, RPS)])
    plsc.subcore_barrier()

    def body(j, r, tail):
        pltpu.make_async_copy(g_hbm.at[srcs[r]], rows[r], gsem[r]).wait()
        pltpu.make_async_copy(dst_hbm.at[base], dsts[r], dsem[r]).wait()
        pltpu.async_copy(rows[r], acc_sh.at[dsts[r]], ssem[r], add=True)
        rp = (r + NRB - 1) % NRB

        @pl.when(j >= 1)
        def _():
            pltpu.make_async_copy(rows[rp], acc_sh.at[dsts[rp]],
                                  ssem[rp]).wait()

        if tail:
            return

        @pl.when(j + 2 < CW)
        def _():
            pltpu.make_async_copy(src_hbm.at[base], srcs[rp], isem[rp]).wait()
            pltpu.async_copy(g_hbm.at[srcs[rp]], rows[rp], gsem[rp])
            pltpu.async_copy(dst_hbm.at[base + j + 2], dsts[rp], dsem[rp])

        @pl.when(j + 3 < CW)
        def _():
            pltpu.async_copy(src_hbm.at[base + j + 3], srcs[r], isem[r])

    for j in range(NRB):
        pltpu.async_copy(src_hbm.at[base + j], srcs[j], isem[j])
    for j in range(2):
        pltpu.async_copy(dst_hbm.at[base + j], dsts[j], dsem[j])
        pltpu.make_async_copy(src_hbm.at[base + j], srcs[j], isem[j]).wait()
        pltpu.async_copy(g_hbm.at[srcs[j]], rows[j], gsem[j])

    @pl.loop(0, CWU // NRB)
    def _(gi):
        for b0 in range(NRB):
            body(gi * NRB + b0, b0, False)

    for j in range(CWU, CW):
        body(j, j % NRB, True)

    pltpu.make_async_copy(rows[(CW - 1) % NRB], acc_sh.at[dsts[0]],
                          ssem[(CW - 1) % NRB]).wait()

    plsc.subcore_barrier()
    pltpu.sync_copy(acc_sh.at[pl.ds(s * RPS, RPS)],
                    out_hbm.at[c, pl.ds(s * RPS, RPS)])


def _dis_body(p0_ref, p1_ref, dis_ref):
    deg = (p0_ref[...] + p1_ref[...])[0, :, 0:1]
    dis = jnp.where(deg > 0, lax.rsqrt(jnp.maximum(deg, 1e-12)), 0.0)
    dis_ref[...] = jnp.broadcast_to(dis, (BLK, C))


def _tc_dis(p):
    return pl.pallas_call(
        _dis_body,
        grid=(GRID,),
        in_specs=[
            pl.BlockSpec((1, BLK, C), lambda i: (0, i, 0)),
            pl.BlockSpec((1, BLK, C), lambda i: (1, i, 0)),
        ],
        out_specs=pl.BlockSpec((BLK, C), lambda i: (i, 0)),
        out_shape=jax.ShapeDtypeStruct((N, C), jnp.float32),
    )(p, p)


def _mm_body(x_ref, w_ref, b_ref, dis_ref, h_ref, g_ref):
    h = lax.dot_general(x_ref[...], w_ref[...], (((1,), (0,)), ((), ())),
                        precision=lax.Precision.HIGHEST,
                        preferred_element_type=jnp.float32)
    h = h + b_ref[...]
    h_ref[...] = h
    g_ref[...] = h * dis_ref[...]


def _tc_matmul_scale(x, w, b, dis):
    return pl.pallas_call(
        _mm_body,
        grid=(GRID,),
        in_specs=[
            pl.BlockSpec((BLK, C), lambda i: (i, 0)),
            pl.BlockSpec((C, C), lambda i: (0, 0)),
            pl.BlockSpec((1, C), lambda i: (0, 0)),
            pl.BlockSpec((BLK, C), lambda i: (i, 0)),
        ],
        out_specs=[
            pl.BlockSpec((BLK, C), lambda i: (i, 0)),
            pl.BlockSpec((BLK, C), lambda i: (i, 0)),
        ],
        out_shape=[
            jax.ShapeDtypeStruct((N, C), jnp.float32),
            jax.ShapeDtypeStruct((N, C), jnp.float32),
        ],
    )(x, w, b, dis)


def _comb_body(p0_ref, p1_ref, dis_ref, h_ref, g_ref):
    dis = dis_ref[...]
    h = (p0_ref[...] + p1_ref[...])[0] * dis
    h_ref[...] = h
    g_ref[...] = h * dis


def _tc_combine(p, dis):
    return pl.pallas_call(
        _comb_body,
        grid=(GRID,),
        in_specs=[
            pl.BlockSpec((1, BLK, C), lambda i: (0, i, 0)),
            pl.BlockSpec((1, BLK, C), lambda i: (1, i, 0)),
            pl.BlockSpec((BLK, C), lambda i: (i, 0)),
        ],
        out_specs=[
            pl.BlockSpec((BLK, C), lambda i: (i, 0)),
            pl.BlockSpec((BLK, C), lambda i: (i, 0)),
        ],
        out_shape=[
            jax.ShapeDtypeStruct((N, C), jnp.float32),
            jax.ShapeDtypeStruct((N, C), jnp.float32),
        ],
    )(p, p, dis)


def _se_mix(h0, h1, h2, h3, se1_ref, se2_ref):
    hops = (h0, h1, h2, h3)
    z = [jnp.mean(h, axis=1, keepdims=True) for h in hops]
    u = [jnp.maximum(sum(z[i] * se1_ref[i, k] for i in range(4)), 0.0)
         for k in range(4)]
    svals = [jax.nn.sigmoid(sum(u[k] * se2_ref[k, j] for k in range(4)))
             for j in range(4)]
    return sum(svals[j] * hops[j] for j in range(4))


def _se0_body(h0_ref, h1_ref, h2_ref, p0_ref, p1_ref, dis_ref, se1_ref,
              se2_ref, g_ref, b_ref, out_ref):
    h3 = (p0_ref[...] + p1_ref[...])[0] * dis_ref[...]
    y = _se_mix(h0_ref[...], h1_ref[...], h2_ref[...], h3, se1_ref, se2_ref)
    mu = jnp.mean(y, axis=1, keepdims=True)
    var = jnp.mean(y * y, axis=1, keepdims=True) - mu * mu
    y = (y - mu) * lax.rsqrt(var + 1e-5) * g_ref[...] + b_ref[...]
    out_ref[...] = jnp.clip(y, 0.0, 6.0)


def _se1_body(h0_ref, h1_ref, h2_ref, p0_ref, p1_ref, dis_ref, se1_ref,
              se2_ref, out_ref):
    h3 = (p0_ref[...] + p1_ref[...])[0] * dis_ref[...]
    y = _se_mix(h0_ref[...], h1_ref[...], h2_ref[...], h3, se1_ref, se2_ref)
    m = jnp.max(y, axis=1, keepdims=True)
    ex = jnp.exp(y - m)
    out_ref[...] = (y - m) - jnp.log(jnp.sum(ex, axis=1, keepdims=True))


_ROWSPEC = pl.BlockSpec((BLK, C), lambda i: (i, 0))
_PSPEC0 = pl.BlockSpec((1, BLK, C), lambda i: (0, i, 0))
_PSPEC1 = pl.BlockSpec((1, BLK, C), lambda i: (1, i, 0))
_SMEMSPEC = pl.BlockSpec(memory_space=pltpu.SMEM)
_VECSPEC = pl.BlockSpec((1, C), lambda i: (0, 0))


def _tc_se0(h0, h1, h2, p, dis, se1, se2, ln_g, ln_b):
    return pl.pallas_call(
        _se0_body,
        grid=(GRID,),
        in_specs=[_ROWSPEC, _ROWSPEC, _ROWSPEC, _PSPEC0, _PSPEC1, _ROWSPEC,
                  _SMEMSPEC, _SMEMSPEC, _VECSPEC, _VECSPEC],
        out_specs=_ROWSPEC,
        out_shape=jax.ShapeDtypeStruct((N, C), jnp.float32),
    )(h0, h1, h2, p, p, dis, se1, se2, ln_g, ln_b)


def _tc_se1(h0, h1, h2, p, dis, se1, se2):
    return pl.pallas_call(
        _se1_body,
        grid=(GRID,),
        in_specs=[_ROWSPEC, _ROWSPEC, _ROWSPEC, _PSPEC0, _PSPEC1, _ROWSPEC,
                  _SMEMSPEC, _SMEMSPEC],
        out_specs=_ROWSPEC,
        out_shape=jax.ShapeDtypeStruct((N, C), jnp.float32),
    )(h0, h1, h2, p, p, dis, se1, se2)


def kernel(n_feat, edge_index, W0, b0, se1_0, se2_0, ln_g, ln_b, W1, b1,
           se1_1, se2_1):
    src = edge_index[0]
    dst = edge_index[1]
    pad = EP - E
    srcp = jnp.concatenate([src, jnp.zeros((pad,), jnp.int32)])
    dstp = jnp.concatenate([dst, jnp.full((pad,), N, jnp.int32)])
    srcR = srcp.reshape(NROWS, CHUNK)
    dstR = dstp.reshape(NROWS, CHUNK)
    zerosA = jnp.zeros((RPS, C), jnp.float32)
    onesN = jnp.ones((N, C), jnp.float32)
    b0r = b0.reshape(1, C)
    b1r = b1.reshape(1, C)
    ln_gr = ln_g.reshape(1, C)
    ln_br = ln_b.reshape(1, C)

    pdeg = _sc_hop(onesN, srcR, dstR, zerosA)
    dis = _tc_dis(pdeg)

    def layer(x, w, b):
        h0, g = _tc_matmul_scale(x, w, b, dis)
        p = _sc_hop(g, srcR, dstR, zerosA)
        h1, g = _tc_combine(p, dis)
        p = _sc_hop(g, srcR, dstR, zerosA)
        h2, g = _tc_combine(p, dis)
        p = _sc_hop(g, srcR, dstR, zerosA)
        return h0, h1, h2, p

    h0, h1, h2, p = layer(n_feat, W0, b0r)
    x1 = _tc_se0(h0, h1, h2, p, dis, se1_0, se2_0, ln_gr, ln_br)
    h0, h1, h2, p = layer(x1, W1, b1r)
    return _tc_se1(h0, h1, h2, p, dis, se1_1, se2_1)


# speedup vs baseline: 1.4252x; 1.0973x over previous
import dataclasses
import functools

import jax
import jax.numpy as jnp
from jax import lax
from jax.experimental import pallas as pl
from jax.experimental.pallas import tpu as pltpu
from jax.experimental.pallas import tpu_sc as plsc

N = 10000
E = 320000
C = 128
KHOPS = 3
NC = 2
NS = 16
NW = NC * NS
CHUNK = 128
CW = 80
EP = NW * CW * CHUNK
NROWS = EP // CHUNK
NACC = 10112
RPS = NACC // NS
BLK = 1000
GRID = N // BLK

_mesh = plsc.VectorSubcoreMesh(core_axis_name="c", subcore_axis_name="s")

_cp_no_layout = pltpu.CompilerParams()
if "needs_layout_passes" in pltpu.CompilerParams.__dataclass_fields__:
    _cp_no_layout = dataclasses.replace(_cp_no_layout,
                                        needs_layout_passes=False)


NRB = 3
CWU = CW - CW % NRB


@functools.partial(
    pl.kernel,
    out_type=jax.ShapeDtypeStruct((NC, NACC, C), jnp.float32),
    mesh=_mesh,
    scratch_types=(
        [pltpu.VMEM((CHUNK,), jnp.int32)] * NRB
        + [pltpu.VMEM((CHUNK,), jnp.int32)] * NRB
        + [pltpu.VMEM((CHUNK, C), jnp.float32)] * NRB
        + [pltpu.SemaphoreType.DMA] * NRB
        + [pltpu.SemaphoreType.DMA] * NRB
        + [pltpu.SemaphoreType.DMA] * NRB
        + [pltpu.SemaphoreType.DMA] * NRB
        + [pltpu.VMEM_SHARED((NACC, C), jnp.float32)]
    ),
)
def _sc_hop(g_hbm, src_hbm, dst_hbm, zeros_hbm, out_hbm, *bufs):
    srcs = bufs[0:NRB]
    dsts = bufs[NRB:2 * NRB]
    rows = bufs[2 * NRB:3 * NRB]
    isem = bufs[3 * NRB:4 * NRB]
    dsem = bufs[4 * NRB:5 * NRB]
    gsem = bufs[5 * NRB:6 * NRB]
    ssem = bufs[6 * NRB:7 * NRB]
    acc_sh = bufs[7 * NRB]
    c = lax.axis_index("c")
    s = lax.axis_index("s")
    w = c * NS + s
    base = w * CW
    pltpu.sync_copy(zeros_hbm, acc_sh.at[pl.ds(s * RPS, RPS)])
    plsc.subcore_barrier()

    def body(j, r, tail):
        pltpu.make_async_copy(g_hbm.at[srcs[r]], rows[r], gsem[r]).wait()
        pltpu.make_async_copy(dst_hbm.at[base], dsts[r], dsem[r]).wait()
        pltpu.async_copy(rows[r], acc_sh.at[dsts[r]], ssem[r], add=True)
        rp = (r + NRB - 1) % NRB

        @pl.when(j >= 1)
        def _():
            pltpu.make_async_copy(rows[rp], acc_sh.at[dsts[rp]],
                                  ssem[rp]).wait()

        if tail:
            return

        @pl.when(j + 2 < CW)
        def _():
            pltpu.make_async_copy(src_hbm.at[base], srcs[rp], isem[rp]).wait()
            pltpu.async_copy(g_hbm.at[srcs[rp]], rows[rp], gsem[rp])
            pltpu.async_copy(dst_hbm.at[base + j + 2], dsts[rp], dsem[rp])

        @pl.when(j + 3 < CW)
        def _():
            pltpu.async_copy(src_hbm.at[base + j + 3], srcs[r], isem[r])

    for j in range(NRB):
        pltpu.async_copy(src_hbm.at[base + j], srcs[j], isem[j])
    for j in range(2):
        pltpu.async_copy(dst_hbm.at[base + j], dsts[j], dsem[j])
        pltpu.make_async_copy(src_hbm.at[base + j], srcs[j], isem[j]).wait()
        pltpu.async_copy(g_hbm.at[srcs[j]], rows[j], gsem[j])

    @pl.loop(0, CWU // NRB)
    def _(gi):
        for b0 in range(NRB):
            body(gi * NRB + b0, b0, False)

    for j in range(CWU, CW):
        body(j, j % NRB, True)

    pltpu.make_async_copy(rows[(CW - 1) % NRB], acc_sh.at[dsts[0]],
                          ssem[(CW - 1) % NRB]).wait()

    plsc.subcore_barrier()
    pltpu.sync_copy(acc_sh.at[pl.ds(s * RPS, RPS)],
                    out_hbm.at[c, pl.ds(s * RPS, RPS)])


DROWS = 80


@functools.partial(
    pl.kernel,
    out_type=jax.ShapeDtypeStruct((NC, DROWS, C), jnp.float32),
    mesh=_mesh,
    scratch_types=[
        pltpu.VMEM((CHUNK,), jnp.int32),
        pltpu.VMEM((CHUNK,), jnp.int32),
        pltpu.VMEM((DROWS, C), jnp.float32),
        pltpu.VMEM((DROWS,), jnp.int32),
        pltpu.VMEM_SHARED((DROWS, C), jnp.float32),
        pltpu.SemaphoreType.DMA,
        pltpu.SemaphoreType.DMA,
        pltpu.SemaphoreType.DMA,
    ],
    compiler_params=_cp_no_layout,
)
def _sc_deg(dst_hbm, id_hbm, zeros_hbm, out_hbm, dstA, dstB, hist, idv,
            acc_sh, semA, semB, sem):
    c = lax.axis_index("c")
    s = lax.axis_index("s")
    w = c * NS + s
    base = w * CW
    pltpu.sync_copy(zeros_hbm.at[pl.ds(0, DROWS)], hist)
    pltpu.sync_copy(id_hbm, idv)

    @pl.when(s == 0)
    def _():
        pltpu.sync_copy(zeros_hbm.at[pl.ds(0, DROWS)], acc_sh)

    plsc.subcore_barrier()

    pltpu.async_copy(dst_hbm.at[base], dstA, semA)
    pltpu.async_copy(dst_hbm.at[base + 1], dstB, semB)
    ones = jnp.full((16,), 1.0, jnp.float32)

    def count(slot):
        for g16 in range(CHUNK // 16):
            d = slot[pl.ds(g16 * 16, 16)]
            row = lax.shift_right_logical(d, 7)
            col = jnp.bitwise_and(d, 127)
            plsc.addupdate_scatter(hist, [row, col], ones)

    @pl.loop(0, CW // 2)
    def _(i):
        j = base + i * 2
        pltpu.make_async_copy(dst_hbm.at[j], dstA, semA).wait()
        count(dstA)

        @pl.when(i * 2 + 2 < CW)
        def _():
            pltpu.async_copy(dst_hbm.at[j + 2], dstA, semA)

        pltpu.make_async_copy(dst_hbm.at[j + 1], dstB, semB).wait()
        count(dstB)

        @pl.when(i * 2 + 3 < CW)
        def _():
            pltpu.async_copy(dst_hbm.at[j + 3], dstB, semB)

    pltpu.async_copy(hist, acc_sh.at[idv], sem, add=True).wait()
    plsc.subcore_barrier()

    @pl.when(s == 0)
    def _():
        pltpu.sync_copy(acc_sh, out_hbm.at[c])


def _dis_body(p0_ref, p1_ref, dis_ref):
    deg = p0_ref[...] + p1_ref[...]
    dis = jnp.where(deg > 0, lax.rsqrt(jnp.maximum(deg, 1e-12)), 0.0)
    dis_ref[...] = jnp.broadcast_to(dis, (BLK, C))


def _tc_dis(p0v, p1v):
    return pl.pallas_call(
        _dis_body,
        grid=(GRID,),
        in_specs=[
            pl.BlockSpec((BLK, 1), lambda i: (i, 0)),
            pl.BlockSpec((BLK, 1), lambda i: (i, 0)),
        ],
        out_specs=pl.BlockSpec((BLK, C), lambda i: (i, 0)),
        out_shape=jax.ShapeDtypeStruct((N, C), jnp.float32),
    )(p0v, p1v)


def _mm_body(x_ref, w_ref, b_ref, dis_ref, h_ref, g_ref):
    h = lax.dot_general(x_ref[...], w_ref[...], (((1,), (0,)), ((), ())),
                        precision=lax.Precision.HIGHEST,
                        preferred_element_type=jnp.float32)
    h = h + b_ref[...]
    h_ref[...] = h
    g_ref[...] = h * dis_ref[...]


def _tc_matmul_scale(x, w, b, dis):
    return pl.pallas_call(
        _mm_body,
        grid=(GRID,),
        in_specs=[
            pl.BlockSpec((BLK, C), lambda i: (i, 0)),
            pl.BlockSpec((C, C), lambda i: (0, 0)),
            pl.BlockSpec((1, C), lambda i: (0, 0)),
            pl.BlockSpec((BLK, C), lambda i: (i, 0)),
        ],
        out_specs=[
            pl.BlockSpec((BLK, C), lambda i: (i, 0)),
            pl.BlockSpec((BLK, C), lambda i: (i, 0)),
        ],
        out_shape=[
            jax.ShapeDtypeStruct((N, C), jnp.float32),
            jax.ShapeDtypeStruct((N, C), jnp.float32),
        ],
    )(x, w, b, dis)


def _comb_body(p0_ref, p1_ref, dis_ref, h_ref, g_ref):
    dis = dis_ref[...]
    h = (p0_ref[...] + p1_ref[...])[0] * dis
    h_ref[...] = h
    g_ref[...] = h * dis


def _tc_combine(p, dis):
    return pl.pallas_call(
        _comb_body,
        grid=(GRID,),
        in_specs=[
            pl.BlockSpec((1, BLK, C), lambda i: (0, i, 0)),
            pl.BlockSpec((1, BLK, C), lambda i: (1, i, 0)),
            pl.BlockSpec((BLK, C), lambda i: (i, 0)),
        ],
        out_specs=[
            pl.BlockSpec((BLK, C), lambda i: (i, 0)),
            pl.BlockSpec((BLK, C), lambda i: (i, 0)),
        ],
        out_shape=[
            jax.ShapeDtypeStruct((N, C), jnp.float32),
            jax.ShapeDtypeStruct((N, C), jnp.float32),
        ],
    )(p, p, dis)


def _se_mix(h0, h1, h2, h3, se1_ref, se2_ref):
    hops = (h0, h1, h2, h3)
    z = [jnp.mean(h, axis=1, keepdims=True) for h in hops]
    u = [jnp.maximum(sum(z[i] * se1_ref[i, k] for i in range(4)), 0.0)
         for k in range(4)]
    svals = [jax.nn.sigmoid(sum(u[k] * se2_ref[k, j] for k in range(4)))
             for j in range(4)]
    return sum(svals[j] * hops[j] for j in range(4))


def _se0_body(h0_ref, h1_ref, h2_ref, p0_ref, p1_ref, dis_ref, se1_ref,
              se2_ref, g_ref, b_ref, out_ref):
    h3 = (p0_ref[...] + p1_ref[...])[0] * dis_ref[...]
    y = _se_mix(h0_ref[...], h1_ref[...], h2_ref[...], h3, se1_ref, se2_ref)
    mu = jnp.mean(y, axis=1, keepdims=True)
    var = jnp.mean(y * y, axis=1, keepdims=True) - mu * mu
    y = (y - mu) * lax.rsqrt(var + 1e-5) * g_ref[...] + b_ref[...]
    out_ref[...] = jnp.clip(y, 0.0, 6.0)


def _se1_body(h0_ref, h1_ref, h2_ref, p0_ref, p1_ref, dis_ref, se1_ref,
              se2_ref, out_ref):
    h3 = (p0_ref[...] + p1_ref[...])[0] * dis_ref[...]
    y = _se_mix(h0_ref[...], h1_ref[...], h2_ref[...], h3, se1_ref, se2_ref)
    m = jnp.max(y, axis=1, keepdims=True)
    ex = jnp.exp(y - m)
    out_ref[...] = (y - m) - jnp.log(jnp.sum(ex, axis=1, keepdims=True))


_ROWSPEC = pl.BlockSpec((BLK, C), lambda i: (i, 0))
_PSPEC0 = pl.BlockSpec((1, BLK, C), lambda i: (0, i, 0))
_PSPEC1 = pl.BlockSpec((1, BLK, C), lambda i: (1, i, 0))
_SMEMSPEC = pl.BlockSpec(memory_space=pltpu.SMEM)
_VECSPEC = pl.BlockSpec((1, C), lambda i: (0, 0))


def _tc_se0(h0, h1, h2, p, dis, se1, se2, ln_g, ln_b):
    return pl.pallas_call(
        _se0_body,
        grid=(GRID,),
        in_specs=[_ROWSPEC, _ROWSPEC, _ROWSPEC, _PSPEC0, _PSPEC1, _ROWSPEC,
                  _SMEMSPEC, _SMEMSPEC, _VECSPEC, _VECSPEC],
        out_specs=_ROWSPEC,
        out_shape=jax.ShapeDtypeStruct((N, C), jnp.float32),
    )(h0, h1, h2, p, p, dis, se1, se2, ln_g, ln_b)


def _tc_se1(h0, h1, h2, p, dis, se1, se2):
    return pl.pallas_call(
        _se1_body,
        grid=(GRID,),
        in_specs=[_ROWSPEC, _ROWSPEC, _ROWSPEC, _PSPEC0, _PSPEC1, _ROWSPEC,
                  _SMEMSPEC, _SMEMSPEC],
        out_specs=_ROWSPEC,
        out_shape=jax.ShapeDtypeStruct((N, C), jnp.float32),
    )(h0, h1, h2, p, p, dis, se1, se2)


def kernel(n_feat, edge_index, W0, b0, se1_0, se2_0, ln_g, ln_b, W1, b1,
           se1_1, se2_1):
    src = edge_index[0]
    dst = edge_index[1]
    pad = EP - E
    srcp = jnp.concatenate([src, jnp.zeros((pad,), jnp.int32)])
    dstp = jnp.concatenate([dst, jnp.full((pad,), N, jnp.int32)])
    srcR = srcp.reshape(NROWS, CHUNK)
    dstR = dstp.reshape(NROWS, CHUNK)
    zerosA = jnp.zeros((RPS, C), jnp.float32)
    id80 = jnp.arange(DROWS, dtype=jnp.int32)
    b0r = b0.reshape(1, C)
    b1r = b1.reshape(1, C)
    ln_gr = ln_g.reshape(1, C)
    ln_br = ln_b.reshape(1, C)

    pdeg = _sc_deg(dstR, id80, zerosA)
    p0v = pdeg[0].reshape(DROWS * C)[:N].reshape(N, 1)
    p1v = pdeg[1].reshape(DROWS * C)[:N].reshape(N, 1)
    dis = _tc_dis(p0v, p1v)

    def layer(x, w, b):
        h0, g = _tc_matmul_scale(x, w, b, dis)
        p = _sc_hop(g, srcR, dstR, zerosA)
        h1, g = _tc_combine(p, dis)
        p = _sc_hop(g, srcR, dstR, zerosA)
        h2, g = _tc_combine(p, dis)
        p = _sc_hop(g, srcR, dstR, zerosA)
        return h0, h1, h2, p

    h0, h1, h2, p = layer(n_feat, W0, b0r)
    x1 = _tc_se0(h0, h1, h2, p, dis, se1_0, se2_0, ln_gr, ln_br)
    h0, h1, h2, p = layer(x1, W1, b1r)
    return _tc_se1(h0, h1, h2, p, dis, se1_1, se2_1)
